# baseline (speedup 1.0000x reference)
import functools

import jax
import jax.numpy as jnp
from jax import lax
from jax.experimental import pallas as pl
from jax.experimental.pallas import tpu as pltpu
from jax.experimental.pallas import tpu_sc as plsc

N = 10000
E = 320000
D = 128
G = 64
NW = 32
EPW = E // NW
CH = 80
NCH = EPW // CH
RPT = N // 16
BLK = 2000
NBLK = N // BLK

def _sc_mesh():
    return plsc.VectorSubcoreMesh(core_axis_name="c", subcore_axis_name="s",
                                  num_cores=2, num_subcores=16)


def _k1_body(x_ref, w1_ref, b_ref, y_ref, cnt_ref):
    i = pl.program_id(0)
    y_ref[...] = lax.dot_general(
        x_ref[...], w1_ref[...], (((1,), (1,)), ((), ())),
        preferred_element_type=jnp.float32)
    oh = (b_ref[...] == lax.broadcasted_iota(jnp.int32, (1, G), 1)
          ).astype(jnp.float32)
    part = jnp.sum(oh, axis=0, keepdims=True)

    @pl.when(i == 0)
    def _():
        cnt_ref[...] = jnp.zeros_like(cnt_ref)

    cnt_ref[...] += part


def _k1(x, w1, batch2d):
    return pl.pallas_call(
        _k1_body,
        grid=(NBLK,),
        in_specs=[
            pl.BlockSpec((BLK, D), lambda i: (i, 0)),
            pl.BlockSpec((D, D), lambda i: (0, 0)),
            pl.BlockSpec((BLK, 1), lambda i: (i, 0)),
        ],
        out_specs=[
            pl.BlockSpec((BLK, D), lambda i: (i, 0)),
            pl.BlockSpec((1, G), lambda i: (0, 0)),
        ],
        out_shape=[
            jax.ShapeDtypeStruct((N, D), jnp.float32),
            jax.ShapeDtypeStruct((1, G), jnp.float32),
        ],
    )(x, w1, batch2d)


@functools.cache
def _k2_kernel():
    return pl.kernel(
        _k2_body,
        out_type=jax.ShapeDtypeStruct((2, N, D), jnp.float32),
        mesh=_sc_mesh(),
        compiler_params=pltpu.CompilerParams(use_tc_tiling_on_sc=False, needs_layout_passes=False),
        scratch_types=[
            pltpu.VMEM((2, CH), jnp.int32),
            pltpu.VMEM((2, CH), jnp.int32),
            pltpu.VMEM((2 * CH,), jnp.float32),
            pltpu.VMEM((CH, D), jnp.float32),
            pltpu.VMEM((CH, D), jnp.float32),
            pltpu.VMEM((CH, D), jnp.float32),
            pltpu.VMEM((CH, D), jnp.float32),
            pltpu.VMEM_SHARED((N, D), jnp.float32),
        ] + [pltpu.SemaphoreType.DMA] * 10,
    )


def _k2(*args):
    return _k2_kernel()(*args)


def _k2_body(y_hbm, src_hbm, dst_hbm, ew_hbm, out_hbm,
             srcr, dstr, ewr, rb0, rb1, ob0, ob1, agg_sh,
             gs0, gs1, ss0, ss1, ps0, ps1, pd0, pd1, pe0, pe1):
    c = lax.axis_index("c")
    s = lax.axis_index("s")
    w = s * 2 + c

    z16 = jnp.zeros((16,), jnp.float32)

    def zrow(i, carry):
        for f in range(D // 16):
            rb0[i, pl.ds(f * 16, 16)] = z16
        return carry

    lax.fori_loop(0, 25, zrow, 0)
    for j in range(RPT // 25):
        pltpu.sync_copy(rb0.at[pl.ds(0, 25)],
                        agg_sh.at[pl.ds(s * RPT + j * 25, 25)])
    plsc.subcore_barrier()

    rbufs = (rb0, rb1)
    obufs = (ob0, ob1)
    gsems = (gs0, gs1)
    ssems = (ss0, ss1)
    srcsems = (ps0, ps1)
    dstsems = (pd0, pd1)
    ewsems = (pe0, pe1)

    for b in range(2):
        pltpu.async_copy(src_hbm.at[w, b], srcr.at[b], srcsems[b])
        pltpu.async_copy(dst_hbm.at[w, b], dstr.at[b], dstsems[b])
        pltpu.async_copy(ew_hbm.at[w, pl.ds(b * CH, CH)],
                         ewr.at[pl.ds(b * CH, CH)], ewsems[b])
    pltpu.make_async_copy(src_hbm.at[w, 0], srcr.at[0], srcsems[0]).wait()
    pltpu.async_copy(y_hbm.at[srcr.at[0]], rb0, gs0)

    def body(i, b):
        rb, ob = rbufs[b], obufs[b]
        pltpu.make_async_copy(y_hbm.at[srcr.at[b]], rb, gsems[b]).wait()
        @pl.when(i + 2 < NCH)
        def _():
            pltpu.async_copy(src_hbm.at[w, i + 2], srcr.at[b], srcsems[b])
        @pl.when(i + 1 < NCH)
        def _():
            pltpu.make_async_copy(src_hbm.at[w, i + 1], srcr.at[1 - b],
                                  srcsems[1 - b]).wait()
            pltpu.async_copy(y_hbm.at[srcr.at[1 - b]], rbufs[1 - b],
                             gsems[1 - b])
        @pl.when(i >= 2)
        def _():
            pltpu.make_async_copy(ob, agg_sh.at[dstr.at[b]], ssems[b]).wait()
            pltpu.async_copy(dst_hbm.at[w, i], dstr.at[b], dstsems[b])
        pltpu.make_async_copy(ew_hbm.at[w, pl.ds(b * CH, CH)],
                              ewr.at[pl.ds(b * CH, CH)], ewsems[b]).wait()

        def scale(e, carry2):
            idx = lax.broadcast(b * CH + e, (16,))
            wv = plsc.load_gather(ewr, [idx])
            for f in range(D // 16):
                ob[e, pl.ds(f * 16, 16)] = rb[e, pl.ds(f * 16, 16)] * wv
            return carry2

        lax.fori_loop(0, CH, scale, 0)
        pltpu.make_async_copy(dst_hbm.at[w, i], dstr.at[b], dstsems[b]).wait()
        pltpu.async_copy(ob, agg_sh.at[dstr.at[b]], ssems[b], add=True)
        @pl.when(i + 2 < NCH)
        def _():
            pltpu.async_copy(ew_hbm.at[w, pl.ds((i + 2) * CH, CH)],
                             ewr.at[pl.ds(b * CH, CH)], ewsems[b])

    def outer(j, carry):
        for b in range(2):
            body(2 * j + b, b)
        return carry

    lax.fori_loop(0, NCH // 2, outer, 0)
    body(jnp.int32(NCH - 1), 0)
    pltpu.make_async_copy(ob1, agg_sh.at[dstr.at[1]], ssems[1]).wait()
    pltpu.make_async_copy(ob0, agg_sh.at[dstr.at[0]], ssems[0]).wait()
    plsc.subcore_barrier()
    pltpu.sync_copy(agg_sh.at[pl.ds(s * RPT, RPT)],
                    out_hbm.at[c, pl.ds(s * RPT, RPT)])


def _k3_body(a0_ref, a1_ref, cnt_ref, b_ref, b1_ref, w2_ref, s_ref):
    a = a0_ref[...] + a1_ref[...]
    oh = (b_ref[...] == lax.broadcasted_iota(jnp.int32, (1, G), 1)
          ).astype(jnp.float32)
    n_each = lax.dot_general(oh, cnt_ref[...], (((1,), (1,)), ((), ())),
                             preferred_element_type=jnp.float32)
    ninv = 1.0 / jnp.maximum(n_each, 1.0)
    h = jax.nn.sigmoid(a * ninv + b1_ref[...])
    s_ref[...] = lax.dot_general(h, w2_ref[...], (((1,), (1,)), ((), ())),
                                 preferred_element_type=jnp.float32)


def _k3(a0, a1, cnt, batch2d, b1r, w2):
    return pl.pallas_call(
        _k3_body,
        grid=(NBLK,),
        in_specs=[
            pl.BlockSpec((BLK, D), lambda i: (i, 0)),
            pl.BlockSpec((BLK, D), lambda i: (i, 0)),
            pl.BlockSpec((1, G), lambda i: (0, 0)),
            pl.BlockSpec((BLK, 1), lambda i: (i, 0)),
            pl.BlockSpec((1, D), lambda i: (0, 0)),
            pl.BlockSpec((1, D), lambda i: (0, 0)),
        ],
        out_specs=pl.BlockSpec((BLK, 1), lambda i: (i, 0)),
        out_shape=jax.ShapeDtypeStruct((N, 1), jnp.float32),
    )(a0, a1, cnt, batch2d, b1r, w2)


@functools.cache
def _k4_kernel():
    return pl.kernel(
        _k4_body,
        out_type=jax.ShapeDtypeStruct((2, N), jnp.float32),
        mesh=_sc_mesh(),
        compiler_params=pltpu.CompilerParams(use_tc_tiling_on_sc=False, needs_layout_passes=False),
        scratch_types=[
            pltpu.VMEM((N,), jnp.float32),
            pltpu.VMEM((EPW,), jnp.int32),
            pltpu.VMEM((NCH, CH), jnp.int32),
            pltpu.VMEM((EPW,), jnp.float32),
            pltpu.VMEM((CH,), jnp.float32),
            pltpu.VMEM((N,), jnp.float32),
            pltpu.VMEM_SHARED((N,), jnp.float32),
        ],
    )


def _k4(*args):
    return _k4_kernel()(*args)


def _k4_body(s_hbm, srcf_hbm, dst_hbm, ew_hbm, out_hbm,
        sb, srcb, dstb, ewb, vb, zb, t_sh):
    c = lax.axis_index("c")
    s = lax.axis_index("s")
    w = s * 2 + c
    pltpu.sync_copy(s_hbm, sb)
    pltpu.sync_copy(srcf_hbm.at[w], srcb)
    pltpu.sync_copy(dst_hbm.at[w], dstb)
    pltpu.sync_copy(ew_hbm.at[w], ewb)

    @pl.when(s == 0)
    def _():
        z16 = jnp.zeros((16,), jnp.float32)

        def zr(i, carry):
            zb[pl.ds(i * 16, 16)] = z16
            return carry

        lax.fori_loop(0, N // 16, zr, 0)
        pltpu.sync_copy(zb, t_sh)

    plsc.subcore_barrier()

    def chunk(i, carry):
        for g in range(CH // 16):
            idx16 = srcb[pl.ds(i * CH + g * 16, 16)]
            sv = plsc.load_gather(sb, [idx16])
            wv = ewb[pl.ds(i * CH + g * 16, 16)]
            vb[pl.ds(g * 16, 16)] = sv * wv
        pltpu.sync_copy(vb, t_sh.at[dstb.at[i]], add=True)
        return carry

    lax.fori_loop(0, NCH, chunk, 0)
    plsc.subcore_barrier()

    @pl.when(s == 0)
    def _():
        pltpu.sync_copy(t_sh, out_hbm.at[c])


def _k5_body(t0_ref, t1_ref, b_ref, cnt_ref, b2_ref, o_ref):
    t = t0_ref[...] + t1_ref[...]
    oh = (b_ref[...] == lax.broadcasted_iota(jnp.int32, (1, G), 1)
          ).astype(jnp.float32)
    acc = lax.dot_general(t, oh, (((1,), (0,)), ((), ())),
                          preferred_element_type=jnp.float32)
    cnt = cnt_ref[...]
    o_ref[...] = acc / jnp.maximum(cnt * cnt, 1.0) + b2_ref[...]


def _k5(t0, t1, batch2d, cnt, b2r):
    return pl.pallas_call(
        _k5_body,
        out_shape=jax.ShapeDtypeStruct((1, G), jnp.float32),
    )(t0, t1, batch2d, cnt, b2r)


def kernel(x, edge_index, edge_weight, batch, W1, b1, W2, b2):
    src = edge_index[0].astype(jnp.int32)
    dst = edge_index[1].astype(jnp.int32)
    batch2d = batch.astype(jnp.int32).reshape(N, 1)
    src_r = src.reshape(NW, NCH, CH)
    dst_r = dst.reshape(NW, NCH, CH)
    srcf = src.reshape(NW, EPW)
    ew_r = edge_weight.reshape(NW, EPW)
    b1r = b1.reshape(1, D)
    b2r = b2.reshape(1, 1)

    y, cnt = _k1(x, W1, batch2d)
    agg2 = _k2(y, src_r, dst_r, ew_r)
    s = _k3(agg2[0], agg2[1], cnt, batch2d, b1r, W2)
    t2 = _k4(s.reshape(N), srcf, dst_r, ew_r)
    p = _k5(t2[0].reshape(1, N), t2[1].reshape(1, N), batch2d, cnt, b2r)
    return p.reshape(G, 1)


# speedup vs baseline: 1.9509x; 1.9509x over previous
import functools

import jax
import jax.numpy as jnp
from jax import lax
from jax.experimental import pallas as pl
from jax.experimental.pallas import tpu as pltpu
from jax.experimental.pallas import tpu_sc as plsc

N = 10000
E = 320000
D = 128
G = 64
NW = 32
EPW = E // NW
CH = 80
NCH = EPW // CH
RPT = N // 16
BLK = 2000
NBLK = N // BLK

def _sc_mesh():
    return plsc.VectorSubcoreMesh(core_axis_name="c", subcore_axis_name="s",
                                  num_cores=2, num_subcores=16)


def _k1_body(x_ref, w1_ref, b_ref, y_ref, cnt_ref):
    i = pl.program_id(0)
    y_ref[...] = lax.dot_general(
        x_ref[...], w1_ref[...], (((1,), (1,)), ((), ())),
        preferred_element_type=jnp.float32)
    oh = (b_ref[...] == lax.broadcasted_iota(jnp.int32, (1, G), 1)
          ).astype(jnp.float32)
    part = jnp.sum(oh, axis=0, keepdims=True)

    @pl.when(i == 0)
    def _():
        cnt_ref[...] = jnp.zeros_like(cnt_ref)

    cnt_ref[...] += part


def _k1(x, w1, batch2d):
    return pl.pallas_call(
        _k1_body,
        grid=(NBLK,),
        in_specs=[
            pl.BlockSpec((BLK, D), lambda i: (i, 0)),
            pl.BlockSpec((D, D), lambda i: (0, 0)),
            pl.BlockSpec((BLK, 1), lambda i: (i, 0)),
        ],
        out_specs=[
            pl.BlockSpec((BLK, D), lambda i: (i, 0)),
            pl.BlockSpec((1, G), lambda i: (0, 0)),
        ],
        out_shape=[
            jax.ShapeDtypeStruct((N, D), jnp.float32),
            jax.ShapeDtypeStruct((1, G), jnp.float32),
        ],
    )(x, w1, batch2d)


@functools.cache
def _k2_kernel():
    return pl.kernel(
        _k2_body,
        out_type=jax.ShapeDtypeStruct((2, N, D), jnp.float32),
        mesh=_sc_mesh(),
        compiler_params=pltpu.CompilerParams(use_tc_tiling_on_sc=False, needs_layout_passes=False),
        scratch_types=[
            pltpu.VMEM((2, CH), jnp.int32),
            pltpu.VMEM((2, CH), jnp.int32),
            pltpu.VMEM((2 * CH,), jnp.float32),
            pltpu.VMEM((CH, D), jnp.float32),
            pltpu.VMEM((CH, D), jnp.float32),
            pltpu.VMEM((CH, D), jnp.float32),
            pltpu.VMEM((CH, D), jnp.float32),
            pltpu.VMEM_SHARED((N, D), jnp.float32),
        ] + [pltpu.SemaphoreType.DMA] * 10,
    )


def _k2(*args):
    return _k2_kernel()(*args)


def _k2_body(y_hbm, src_hbm, dst_hbm, ew_hbm, out_hbm,
             srcr, dstr, ewr, rb0, rb1, ob0, ob1, agg_sh,
             gs0, gs1, ss0, ss1, ps0, ps1, pd0, pd1, pe0, pe1):
    c = lax.axis_index("c")
    s = lax.axis_index("s")
    w = s * 2 + c

    z16 = jnp.zeros((16,), jnp.float32)

    def zrow(i, carry):
        for f in range(D // 16):
            rb0[i, pl.ds(f * 16, 16)] = z16
        return carry

    lax.fori_loop(0, 25, zrow, 0)
    for j in range(RPT // 25):
        pltpu.sync_copy(rb0.at[pl.ds(0, 25)],
                        agg_sh.at[pl.ds(s * RPT + j * 25, 25)])
    plsc.subcore_barrier()

    rbufs = (rb0, rb1)
    obufs = (ob0, ob1)
    gsems = (gs0, gs1)
    ssems = (ss0, ss1)
    srcsems = (ps0, ps1)
    dstsems = (pd0, pd1)
    ewsems = (pe0, pe1)

    for b in range(2):
        pltpu.async_copy(src_hbm.at[w, b], srcr.at[b], srcsems[b])
        pltpu.async_copy(dst_hbm.at[w, b], dstr.at[b], dstsems[b])
        pltpu.async_copy(ew_hbm.at[w, pl.ds(b * CH, CH)],
                         ewr.at[pl.ds(b * CH, CH)], ewsems[b])
    pltpu.make_async_copy(src_hbm.at[w, 0], srcr.at[0], srcsems[0]).wait()
    pltpu.async_copy(y_hbm.at[srcr.at[0]], rb0, gs0)

    def body(i, b):
        rb, ob = rbufs[b], obufs[b]
        pltpu.make_async_copy(y_hbm.at[srcr.at[b]], rb, gsems[b]).wait()
        @pl.when(i + 2 < NCH)
        def _():
            pltpu.async_copy(src_hbm.at[w, i + 2], srcr.at[b], srcsems[b])
        @pl.when(i + 1 < NCH)
        def _():
            pltpu.make_async_copy(src_hbm.at[w, i + 1], srcr.at[1 - b],
                                  srcsems[1 - b]).wait()
            pltpu.async_copy(y_hbm.at[srcr.at[1 - b]], rbufs[1 - b],
                             gsems[1 - b])
        @pl.when(i >= 2)
        def _():
            pltpu.make_async_copy(ob, agg_sh.at[dstr.at[b]], ssems[b]).wait()
            pltpu.async_copy(dst_hbm.at[w, i], dstr.at[b], dstsems[b])
        pltpu.make_async_copy(ew_hbm.at[w, pl.ds(b * CH, CH)],
                              ewr.at[pl.ds(b * CH, CH)], ewsems[b]).wait()

        def scale(e):
            idx = lax.broadcast(b * CH + e, (16,))
            wv = plsc.load_gather(ewr, [idx])
            for f in range(D // 16):
                ob[e, pl.ds(f * 16, 16)] = rb[e, pl.ds(f * 16, 16)] * wv

        plsc.parallel_loop(0, CH, 1, unroll=4)(scale)
        pltpu.make_async_copy(dst_hbm.at[w, i], dstr.at[b], dstsems[b]).wait()
        pltpu.async_copy(ob, agg_sh.at[dstr.at[b]], ssems[b], add=True)
        @pl.when(i + 2 < NCH)
        def _():
            pltpu.async_copy(ew_hbm.at[w, pl.ds((i + 2) * CH, CH)],
                             ewr.at[pl.ds(b * CH, CH)], ewsems[b])

    def outer(j, carry):
        for b in range(2):
            body(2 * j + b, b)
        return carry

    lax.fori_loop(0, NCH // 2, outer, 0)
    body(jnp.int32(NCH - 1), 0)
    pltpu.make_async_copy(ob1, agg_sh.at[dstr.at[1]], ssems[1]).wait()
    pltpu.make_async_copy(ob0, agg_sh.at[dstr.at[0]], ssems[0]).wait()
    plsc.subcore_barrier()
    pltpu.sync_copy(agg_sh.at[pl.ds(s * RPT, RPT)],
                    out_hbm.at[c, pl.ds(s * RPT, RPT)])


def _k3_body(a0_ref, a1_ref, cnt_ref, b_ref, b1_ref, w2_ref, s_ref):
    a = a0_ref[...] + a1_ref[...]
    oh = (b_ref[...] == lax.broadcasted_iota(jnp.int32, (1, G), 1)
          ).astype(jnp.float32)
    n_each = lax.dot_general(oh, cnt_ref[...], (((1,), (1,)), ((), ())),
                             preferred_element_type=jnp.float32)
    ninv = 1.0 / jnp.maximum(n_each, 1.0)
    h = jax.nn.sigmoid(a * ninv + b1_ref[...])
    s_ref[...] = lax.dot_general(h, w2_ref[...], (((1,), (1,)), ((), ())),
                                 preferred_element_type=jnp.float32)


def _k3(a0, a1, cnt, batch2d, b1r, w2):
    return pl.pallas_call(
        _k3_body,
        grid=(NBLK,),
        in_specs=[
            pl.BlockSpec((BLK, D), lambda i: (i, 0)),
            pl.BlockSpec((BLK, D), lambda i: (i, 0)),
            pl.BlockSpec((1, G), lambda i: (0, 0)),
            pl.BlockSpec((BLK, 1), lambda i: (i, 0)),
            pl.BlockSpec((1, D), lambda i: (0, 0)),
            pl.BlockSpec((1, D), lambda i: (0, 0)),
        ],
        out_specs=pl.BlockSpec((BLK, 1), lambda i: (i, 0)),
        out_shape=jax.ShapeDtypeStruct((N, 1), jnp.float32),
    )(a0, a1, cnt, batch2d, b1r, w2)


@functools.cache
def _k4_kernel():
    return pl.kernel(
        _k4_body,
        out_type=jax.ShapeDtypeStruct((2, N), jnp.float32),
        mesh=_sc_mesh(),
        compiler_params=pltpu.CompilerParams(use_tc_tiling_on_sc=False, needs_layout_passes=False),
        scratch_types=[
            pltpu.VMEM((N,), jnp.float32),
            pltpu.VMEM((EPW,), jnp.int32),
            pltpu.VMEM((NCH, CH), jnp.int32),
            pltpu.VMEM((EPW,), jnp.float32),
            pltpu.VMEM((CH,), jnp.float32),
            pltpu.VMEM((N,), jnp.float32),
            pltpu.VMEM_SHARED((N,), jnp.float32),
        ],
    )


def _k4(*args):
    return _k4_kernel()(*args)


def _k4_body(s_hbm, srcf_hbm, dst_hbm, ew_hbm, out_hbm,
        sb, srcb, dstb, ewb, vb, zb, t_sh):
    c = lax.axis_index("c")
    s = lax.axis_index("s")
    w = s * 2 + c
    pltpu.sync_copy(s_hbm, sb)
    pltpu.sync_copy(srcf_hbm.at[w], srcb)
    pltpu.sync_copy(dst_hbm.at[w], dstb)
    pltpu.sync_copy(ew_hbm.at[w], ewb)

    @pl.when(s == 0)
    def _():
        z16 = jnp.zeros((16,), jnp.float32)

        def zr(i, carry):
            zb[pl.ds(i * 16, 16)] = z16
            return carry

        lax.fori_loop(0, N // 16, zr, 0)
        pltpu.sync_copy(zb, t_sh)

    plsc.subcore_barrier()

    def chunk(i, carry):
        for g in range(CH // 16):
            idx16 = srcb[pl.ds(i * CH + g * 16, 16)]
            sv = plsc.load_gather(sb, [idx16])
            wv = ewb[pl.ds(i * CH + g * 16, 16)]
            vb[pl.ds(g * 16, 16)] = sv * wv
        pltpu.sync_copy(vb, t_sh.at[dstb.at[i]], add=True)
        return carry

    lax.fori_loop(0, NCH, chunk, 0)
    plsc.subcore_barrier()

    @pl.when(s == 0)
    def _():
        pltpu.sync_copy(t_sh, out_hbm.at[c])


def _k5_body(t0_ref, t1_ref, b_ref, cnt_ref, b2_ref, o_ref):
    t = t0_ref[...] + t1_ref[...]
    oh = (b_ref[...] == lax.broadcasted_iota(jnp.int32, (1, G), 1)
          ).astype(jnp.float32)
    acc = lax.dot_general(t, oh, (((1,), (0,)), ((), ())),
                          preferred_element_type=jnp.float32)
    cnt = cnt_ref[...]
    o_ref[...] = acc / jnp.maximum(cnt * cnt, 1.0) + b2_ref[...]


def _k5(t0, t1, batch2d, cnt, b2r):
    return pl.pallas_call(
        _k5_body,
        out_shape=jax.ShapeDtypeStruct((1, G), jnp.float32),
    )(t0, t1, batch2d, cnt, b2r)


def kernel(x, edge_index, edge_weight, batch, W1, b1, W2, b2):
    src = edge_index[0].astype(jnp.int32)
    dst = edge_index[1].astype(jnp.int32)
    batch2d = batch.astype(jnp.int32).reshape(N, 1)
    src_r = src.reshape(NW, NCH, CH)
    dst_r = dst.reshape(NW, NCH, CH)
    srcf = src.reshape(NW, EPW)
    ew_r = edge_weight.reshape(NW, EPW)
    b1r = b1.reshape(1, D)
    b2r = b2.reshape(1, 1)

    y, cnt = _k1(x, W1, batch2d)
    agg2 = _k2(y, src_r, dst_r, ew_r)
    s = _k3(agg2[0], agg2[1], cnt, batch2d, b1r, W2)
    t2 = _k4(s.reshape(N), srcf, dst_r, ew_r)
    p = _k5(t2[0].reshape(1, N), t2[1].reshape(1, N), batch2d, cnt, b2r)
    return p.reshape(G, 1)


# speedup vs baseline: 1.9511x; 1.0001x over previous
import functools

import jax
import jax.numpy as jnp
from jax import lax
from jax.experimental import pallas as pl
from jax.experimental.pallas import tpu as pltpu
from jax.experimental.pallas import tpu_sc as plsc

N = 10000
E = 320000
D = 128
G = 64
NW = 32
EPW = E // NW
CH = 80
NCH = EPW // CH
RPT = N // 16
BLK = 2000
NBLK = N // BLK

def _sc_mesh():
    return plsc.VectorSubcoreMesh(core_axis_name="c", subcore_axis_name="s",
                                  num_cores=2, num_subcores=16)


def _k1_body(x_ref, w1_ref, b_ref, y_ref, cnt_ref):
    i = pl.program_id(0)
    y_ref[...] = lax.dot_general(
        x_ref[...], w1_ref[...], (((1,), (1,)), ((), ())),
        preferred_element_type=jnp.float32)
    oh = (b_ref[...] == lax.broadcasted_iota(jnp.int32, (1, G), 1)
          ).astype(jnp.float32)
    part = jnp.sum(oh, axis=0, keepdims=True)

    @pl.when(i == 0)
    def _():
        cnt_ref[...] = jnp.zeros_like(cnt_ref)

    cnt_ref[...] += part


def _k1(x, w1, batch2d):
    return pl.pallas_call(
        _k1_body,
        grid=(NBLK,),
        in_specs=[
            pl.BlockSpec((BLK, D), lambda i: (i, 0)),
            pl.BlockSpec((D, D), lambda i: (0, 0)),
            pl.BlockSpec((BLK, 1), lambda i: (i, 0)),
        ],
        out_specs=[
            pl.BlockSpec((BLK, D), lambda i: (i, 0)),
            pl.BlockSpec((1, G), lambda i: (0, 0)),
        ],
        out_shape=[
            jax.ShapeDtypeStruct((N, D), jnp.float32),
            jax.ShapeDtypeStruct((1, G), jnp.float32),
        ],
    )(x, w1, batch2d)


@functools.cache
def _k2_kernel():
    return pl.kernel(
        _k2_body,
        out_type=jax.ShapeDtypeStruct((2, N, D), jnp.float32),
        mesh=_sc_mesh(),
        compiler_params=pltpu.CompilerParams(use_tc_tiling_on_sc=False, needs_layout_passes=False),
        scratch_types=[
            pltpu.VMEM((2, CH), jnp.int32),
            pltpu.VMEM((2, CH), jnp.int32),
            pltpu.VMEM((2 * CH,), jnp.float32),
            pltpu.VMEM((CH, D), jnp.float32),
            pltpu.VMEM((CH, D), jnp.float32),
            pltpu.VMEM((CH, D), jnp.float32),
            pltpu.VMEM((CH, D), jnp.float32),
            pltpu.VMEM_SHARED((N, D), jnp.float32),
        ] + [pltpu.SemaphoreType.DMA] * 10,
    )


def _k2(*args):
    return _k2_kernel()(*args)


def _k2_body(y_hbm, src_hbm, dst_hbm, ew_hbm, out_hbm,
             srcr, dstr, ewr, rb0, rb1, ob0, ob1, agg_sh,
             gs0, gs1, ss0, ss1, ps0, ps1, pd0, pd1, pe0, pe1):
    c = lax.axis_index("c")
    s = lax.axis_index("s")
    w = s * 2 + c

    z16 = jnp.zeros((16,), jnp.float32)

    def zrow(i, carry):
        for f in range(D // 16):
            rb0[i, pl.ds(f * 16, 16)] = z16
        return carry

    lax.fori_loop(0, 25, zrow, 0)
    for j in range(RPT // 25):
        pltpu.sync_copy(rb0.at[pl.ds(0, 25)],
                        agg_sh.at[pl.ds(s * RPT + j * 25, 25)])
    plsc.subcore_barrier()

    rbufs = (rb0, rb1)
    obufs = (ob0, ob1)
    gsems = (gs0, gs1)
    ssems = (ss0, ss1)
    srcsems = (ps0, ps1)
    dstsems = (pd0, pd1)
    ewsems = (pe0, pe1)

    for b in range(2):
        pltpu.async_copy(src_hbm.at[w, b], srcr.at[b], srcsems[b])
        pltpu.async_copy(dst_hbm.at[w, b], dstr.at[b], dstsems[b])
        pltpu.async_copy(ew_hbm.at[w, pl.ds(b * CH, CH)],
                         ewr.at[pl.ds(b * CH, CH)], ewsems[b])
    pltpu.make_async_copy(src_hbm.at[w, 0], srcr.at[0], srcsems[0]).wait()
    pltpu.async_copy(y_hbm.at[srcr.at[0]], rb0, gs0)

    def body(i, b):
        rb, ob = rbufs[b], obufs[b]
        pltpu.make_async_copy(y_hbm.at[srcr.at[b]], rb, gsems[b]).wait()
        @pl.when(i + 2 < NCH)
        def _():
            pltpu.async_copy(src_hbm.at[w, i + 2], srcr.at[b], srcsems[b])
        @pl.when(i + 1 < NCH)
        def _():
            pltpu.make_async_copy(src_hbm.at[w, i + 1], srcr.at[1 - b],
                                  srcsems[1 - b]).wait()
            pltpu.async_copy(y_hbm.at[srcr.at[1 - b]], rbufs[1 - b],
                             gsems[1 - b])
        @pl.when(i >= 2)
        def _():
            pltpu.make_async_copy(ob, agg_sh.at[dstr.at[b]], ssems[b]).wait()
            pltpu.async_copy(dst_hbm.at[w, i], dstr.at[b], dstsems[b])
        pltpu.make_async_copy(ew_hbm.at[w, pl.ds(b * CH, CH)],
                              ewr.at[pl.ds(b * CH, CH)], ewsems[b]).wait()

        def scale(e):
            idx = lax.broadcast(b * CH + e, (16,))
            wv = plsc.load_gather(ewr, [idx])
            for f in range(D // 16):
                ob[e, pl.ds(f * 16, 16)] = rb[e, pl.ds(f * 16, 16)] * wv

        plsc.parallel_loop(0, CH, 1, unroll=8)(scale)
        pltpu.make_async_copy(dst_hbm.at[w, i], dstr.at[b], dstsems[b]).wait()
        pltpu.async_copy(ob, agg_sh.at[dstr.at[b]], ssems[b], add=True)
        @pl.when(i + 2 < NCH)
        def _():
            pltpu.async_copy(ew_hbm.at[w, pl.ds((i + 2) * CH, CH)],
                             ewr.at[pl.ds(b * CH, CH)], ewsems[b])

    def outer(j, carry):
        for b in range(2):
            body(2 * j + b, b)
        return carry

    lax.fori_loop(0, NCH // 2, outer, 0)
    body(jnp.int32(NCH - 1), 0)
    pltpu.make_async_copy(ob1, agg_sh.at[dstr.at[1]], ssems[1]).wait()
    pltpu.make_async_copy(ob0, agg_sh.at[dstr.at[0]], ssems[0]).wait()
    plsc.subcore_barrier()
    pltpu.sync_copy(agg_sh.at[pl.ds(s * RPT, RPT)],
                    out_hbm.at[c, pl.ds(s * RPT, RPT)])


def _k3_body(a0_ref, a1_ref, cnt_ref, b_ref, b1_ref, w2_ref, s_ref):
    a = a0_ref[...] + a1_ref[...]
    oh = (b_ref[...] == lax.broadcasted_iota(jnp.int32, (1, G), 1)
          ).astype(jnp.float32)
    n_each = lax.dot_general(oh, cnt_ref[...], (((1,), (1,)), ((), ())),
                             preferred_element_type=jnp.float32)
    ninv = 1.0 / jnp.maximum(n_each, 1.0)
    h = jax.nn.sigmoid(a * ninv + b1_ref[...])
    s_ref[...] = lax.dot_general(h, w2_ref[...], (((1,), (1,)), ((), ())),
                                 preferred_element_type=jnp.float32)


def _k3(a0, a1, cnt, batch2d, b1r, w2):
    return pl.pallas_call(
        _k3_body,
        grid=(NBLK,),
        in_specs=[
            pl.BlockSpec((BLK, D), lambda i: (i, 0)),
            pl.BlockSpec((BLK, D), lambda i: (i, 0)),
            pl.BlockSpec((1, G), lambda i: (0, 0)),
            pl.BlockSpec((BLK, 1), lambda i: (i, 0)),
            pl.BlockSpec((1, D), lambda i: (0, 0)),
            pl.BlockSpec((1, D), lambda i: (0, 0)),
        ],
        out_specs=pl.BlockSpec((BLK, 1), lambda i: (i, 0)),
        out_shape=jax.ShapeDtypeStruct((N, 1), jnp.float32),
    )(a0, a1, cnt, batch2d, b1r, w2)


@functools.cache
def _k4_kernel():
    return pl.kernel(
        _k4_body,
        out_type=jax.ShapeDtypeStruct((2, N), jnp.float32),
        mesh=_sc_mesh(),
        compiler_params=pltpu.CompilerParams(use_tc_tiling_on_sc=False, needs_layout_passes=False),
        scratch_types=[
            pltpu.VMEM((N,), jnp.float32),
            pltpu.VMEM((EPW,), jnp.int32),
            pltpu.VMEM((NCH, CH), jnp.int32),
            pltpu.VMEM((EPW,), jnp.float32),
            pltpu.VMEM((CH,), jnp.float32),
            pltpu.VMEM((N,), jnp.float32),
            pltpu.VMEM_SHARED((N,), jnp.float32),
        ],
    )


def _k4(*args):
    return _k4_kernel()(*args)


def _k4_body(s_hbm, srcf_hbm, dst_hbm, ew_hbm, out_hbm,
        sb, srcb, dstb, ewb, vb, zb, t_sh):
    c = lax.axis_index("c")
    s = lax.axis_index("s")
    w = s * 2 + c
    pltpu.sync_copy(s_hbm, sb)
    pltpu.sync_copy(srcf_hbm.at[w], srcb)
    pltpu.sync_copy(dst_hbm.at[w], dstb)
    pltpu.sync_copy(ew_hbm.at[w], ewb)

    @pl.when(s == 0)
    def _():
        z16 = jnp.zeros((16,), jnp.float32)

        def zr(i, carry):
            zb[pl.ds(i * 16, 16)] = z16
            return carry

        lax.fori_loop(0, N // 16, zr, 0)
        pltpu.sync_copy(zb, t_sh)

    plsc.subcore_barrier()

    def chunk(i, carry):
        for g in range(CH // 16):
            idx16 = srcb[pl.ds(i * CH + g * 16, 16)]
            sv = plsc.load_gather(sb, [idx16])
            wv = ewb[pl.ds(i * CH + g * 16, 16)]
            vb[pl.ds(g * 16, 16)] = sv * wv
        pltpu.sync_copy(vb, t_sh.at[dstb.at[i]], add=True)
        return carry

    lax.fori_loop(0, NCH, chunk, 0)
    plsc.subcore_barrier()

    @pl.when(s == 0)
    def _():
        pltpu.sync_copy(t_sh, out_hbm.at[c])


def _k5_body(t0_ref, t1_ref, b_ref, cnt_ref, b2_ref, o_ref):
    t = t0_ref[...] + t1_ref[...]
    oh = (b_ref[...] == lax.broadcasted_iota(jnp.int32, (1, G), 1)
          ).astype(jnp.float32)
    acc = lax.dot_general(t, oh, (((1,), (0,)), ((), ())),
                          preferred_element_type=jnp.float32)
    cnt = cnt_ref[...]
    o_ref[...] = acc / jnp.maximum(cnt * cnt, 1.0) + b2_ref[...]


def _k5(t0, t1, batch2d, cnt, b2r):
    return pl.pallas_call(
        _k5_body,
        out_shape=jax.ShapeDtypeStruct((1, G), jnp.float32),
    )(t0, t1, batch2d, cnt, b2r)


def kernel(x, edge_index, edge_weight, batch, W1, b1, W2, b2):
    src = edge_index[0].astype(jnp.int32)
    dst = edge_index[1].astype(jnp.int32)
    batch2d = batch.astype(jnp.int32).reshape(N, 1)
    src_r = src.reshape(NW, NCH, CH)
    dst_r = dst.reshape(NW, NCH, CH)
    srcf = src.reshape(NW, EPW)
    ew_r = edge_weight.reshape(NW, EPW)
    b1r = b1.reshape(1, D)
    b2r = b2.reshape(1, 1)

    y, cnt = _k1(x, W1, batch2d)
    agg2 = _k2(y, src_r, dst_r, ew_r)
    s = _k3(agg2[0], agg2[1], cnt, batch2d, b1r, W2)
    t2 = _k4(s.reshape(N), srcf, dst_r, ew_r)
    p = _k5(t2[0].reshape(1, N), t2[1].reshape(1, N), batch2d, cnt, b2r)
    return p.reshape(G, 1)


# speedup vs baseline: 2.0143x; 1.0324x over previous
import functools

import jax
import jax.numpy as jnp
from jax import lax
from jax.experimental import pallas as pl
from jax.experimental.pallas import tpu as pltpu
from jax.experimental.pallas import tpu_sc as plsc

N = 10000
E = 320000
D = 128
G = 64
NW = 32
EPW = E // NW
CH = 80
NCH = EPW // CH
RPT = N // 16
BLK = 2000
NBLK = N // BLK

def _sc_mesh():
    return plsc.VectorSubcoreMesh(core_axis_name="c", subcore_axis_name="s",
                                  num_cores=2, num_subcores=16)


@functools.cache
def _k2_kernel():
    return pl.kernel(
        _k2_body,
        out_type=jax.ShapeDtypeStruct((2, N, D), jnp.float32),
        mesh=_sc_mesh(),
        compiler_params=pltpu.CompilerParams(use_tc_tiling_on_sc=False, needs_layout_passes=False),
        scratch_types=[
            pltpu.VMEM((2, CH), jnp.int32),
            pltpu.VMEM((2, CH), jnp.int32),
            pltpu.VMEM((2 * CH,), jnp.float32),
            pltpu.VMEM((CH, D), jnp.float32),
            pltpu.VMEM((CH, D), jnp.float32),
            pltpu.VMEM((CH, D), jnp.float32),
            pltpu.VMEM((CH, D), jnp.float32),
            pltpu.VMEM_SHARED((N, D), jnp.float32),
        ] + [pltpu.SemaphoreType.DMA] * 10,
    )


def _k2(*args):
    return _k2_kernel()(*args)


def _k2_body(y_hbm, src_hbm, dst_hbm, ew_hbm, out_hbm,
             srcr, dstr, ewr, rb0, rb1, ob0, ob1, agg_sh,
             gs0, gs1, ss0, ss1, ps0, ps1, pd0, pd1, pe0, pe1):
    c = lax.axis_index("c")
    s = lax.axis_index("s")
    w = s * 2 + c

    z16 = jnp.zeros((16,), jnp.float32)

    def zrow(i, carry):
        for f in range(D // 16):
            rb0[i, pl.ds(f * 16, 16)] = z16
        return carry

    lax.fori_loop(0, 25, zrow, 0)
    for j in range(RPT // 25):
        pltpu.sync_copy(rb0.at[pl.ds(0, 25)],
                        agg_sh.at[pl.ds(s * RPT + j * 25, 25)])
    plsc.subcore_barrier()

    rbufs = (rb0, rb1)
    obufs = (ob0, ob1)
    gsems = (gs0, gs1)
    ssems = (ss0, ss1)
    srcsems = (ps0, ps1)
    dstsems = (pd0, pd1)
    ewsems = (pe0, pe1)

    for b in range(2):
        pltpu.async_copy(src_hbm.at[w, b], srcr.at[b], srcsems[b])
        pltpu.async_copy(dst_hbm.at[w, b], dstr.at[b], dstsems[b])
        pltpu.async_copy(ew_hbm.at[w, pl.ds(b * CH, CH)],
                         ewr.at[pl.ds(b * CH, CH)], ewsems[b])
    pltpu.make_async_copy(src_hbm.at[w, 0], srcr.at[0], srcsems[0]).wait()
    pltpu.async_copy(y_hbm.at[srcr.at[0]], rb0, gs0)

    def body(i, b):
        rb, ob = rbufs[b], obufs[b]
        pltpu.make_async_copy(y_hbm.at[srcr.at[b]], rb, gsems[b]).wait()
        @pl.when(i + 2 < NCH)
        def _():
            pltpu.async_copy(src_hbm.at[w, i + 2], srcr.at[b], srcsems[b])
        @pl.when(i + 1 < NCH)
        def _():
            pltpu.make_async_copy(src_hbm.at[w, i + 1], srcr.at[1 - b],
                                  srcsems[1 - b]).wait()
            pltpu.async_copy(y_hbm.at[srcr.at[1 - b]], rbufs[1 - b],
                             gsems[1 - b])
        @pl.when(i >= 2)
        def _():
            pltpu.make_async_copy(ob, agg_sh.at[dstr.at[b]], ssems[b]).wait()
            pltpu.async_copy(dst_hbm.at[w, i], dstr.at[b], dstsems[b])
        pltpu.make_async_copy(ew_hbm.at[w, pl.ds(b * CH, CH)],
                              ewr.at[pl.ds(b * CH, CH)], ewsems[b]).wait()

        def scale(e):
            idx = lax.broadcast(b * CH + e, (16,))
            wv = plsc.load_gather(ewr, [idx])
            for f in range(D // 16):
                ob[e, pl.ds(f * 16, 16)] = rb[e, pl.ds(f * 16, 16)] * wv

        plsc.parallel_loop(0, CH, 1, unroll=4)(scale)
        pltpu.make_async_copy(dst_hbm.at[w, i], dstr.at[b], dstsems[b]).wait()
        pltpu.async_copy(ob, agg_sh.at[dstr.at[b]], ssems[b], add=True)
        @pl.when(i + 2 < NCH)
        def _():
            pltpu.async_copy(ew_hbm.at[w, pl.ds((i + 2) * CH, CH)],
                             ewr.at[pl.ds(b * CH, CH)], ewsems[b])

    def outer(j, carry):
        for b in range(2):
            body(2 * j + b, b)
        return carry

    lax.fori_loop(0, NCH // 2, outer, 0)
    body(jnp.int32(NCH - 1), 0)
    pltpu.make_async_copy(ob1, agg_sh.at[dstr.at[1]], ssems[1]).wait()
    pltpu.make_async_copy(ob0, agg_sh.at[dstr.at[0]], ssems[0]).wait()
    plsc.subcore_barrier()
    pltpu.sync_copy(agg_sh.at[pl.ds(s * RPT, RPT)],
                    out_hbm.at[c, pl.ds(s * RPT, RPT)])


def _k3_body(a0_ref, a1_ref, b_ref, b1_ref, w1_ref, w2_ref, s_ref, cnt_ref):
    p = pl.program_id(0)
    oh = (b_ref[...] == lax.broadcasted_iota(jnp.int32, (1, G), 1)
          ).astype(jnp.float32)

    @pl.when(p == 0)
    def _():
        @pl.when(pl.program_id(1) == 0)
        def _():
            cnt_ref[...] = jnp.zeros_like(cnt_ref)

        cnt_ref[...] += jnp.sum(oh, axis=0, keepdims=True)

    @pl.when(p == 1)
    def _():
        n_each = lax.dot_general(oh, cnt_ref[...], (((1,), (1,)), ((), ())),
                                 preferred_element_type=jnp.float32,
                                 precision=lax.Precision.HIGHEST)
        ninv = 1.0 / jnp.maximum(n_each, 1.0)
        a = (a0_ref[...] + a1_ref[...]) * ninv
        h = jax.nn.sigmoid(
            lax.dot_general(a, w1_ref[...], (((1,), (1,)), ((), ())),
                            preferred_element_type=jnp.float32,
                            precision=lax.Precision.HIGHEST) + b1_ref[...])
        s_ref[...] = lax.dot_general(h, w2_ref[...], (((1,), (1,)), ((), ())),
                                     preferred_element_type=jnp.float32,
                                     precision=lax.Precision.HIGHEST)


def _k3(a0, a1, batch2d, b1r, w1, w2):
    return pl.pallas_call(
        _k3_body,
        grid=(2, NBLK),
        in_specs=[
            pl.BlockSpec((BLK, D), lambda p, j: (p * j, 0)),
            pl.BlockSpec((BLK, D), lambda p, j: (p * j, 0)),
            pl.BlockSpec((BLK, 1), lambda p, j: (j, 0)),
            pl.BlockSpec((1, D), lambda p, j: (0, 0)),
            pl.BlockSpec((D, D), lambda p, j: (0, 0)),
            pl.BlockSpec((1, D), lambda p, j: (0, 0)),
        ],
        out_specs=[
            pl.BlockSpec((BLK, 1), lambda p, j: (p * j, 0)),
            pl.BlockSpec((1, G), lambda p, j: (0, 0)),
        ],
        out_shape=[
            jax.ShapeDtypeStruct((N, 1), jnp.float32),
            jax.ShapeDtypeStruct((1, G), jnp.float32),
        ],
    )(a0, a1, batch2d, b1r, w1, w2)


@functools.cache
def _k4_kernel():
    return pl.kernel(
        _k4_body,
        out_type=jax.ShapeDtypeStruct((2, G), jnp.float32),
        mesh=_sc_mesh(),
        compiler_params=pltpu.CompilerParams(use_tc_tiling_on_sc=False, needs_layout_passes=False),
        scratch_types=[
            pltpu.VMEM((N,), jnp.float32),
            pltpu.VMEM((N,), jnp.int32),
            pltpu.VMEM((EPW,), jnp.int32),
            pltpu.VMEM((EPW,), jnp.int32),
            pltpu.VMEM((EPW,), jnp.float32),
            pltpu.VMEM((1, CH), jnp.float32),
            pltpu.VMEM((1, CH), jnp.int32),
            pltpu.VMEM_SHARED((G,), jnp.float32),
        ],
    )


def _k4(*args):
    return _k4_kernel()(*args)


def _k4_body(s_hbm, srcf_hbm, dstf_hbm, ew_hbm, batch_hbm, out_hbm,
             sb, bb, srcb, dstb, ewb, vb, gb, pp_sh):
    c = lax.axis_index("c")
    s = lax.axis_index("s")
    w = s * 2 + c
    pltpu.sync_copy(s_hbm, sb)
    pltpu.sync_copy(batch_hbm, bb)
    pltpu.sync_copy(srcf_hbm.at[w], srcb)
    pltpu.sync_copy(dstf_hbm.at[w], dstb)
    pltpu.sync_copy(ew_hbm.at[w], ewb)

    z16 = jnp.zeros((16,), jnp.float32)
    for g in range(G // 16):
        vb[0, pl.ds(g * 16, 16)] = z16

    @pl.when(s == 0)
    def _():
        pltpu.sync_copy(vb.at[0, pl.ds(0, G)], pp_sh)

    plsc.subcore_barrier()

    def chunk(i, carry):
        for g in range(CH // 16):
            o = i * CH + g * 16
            src16 = srcb[pl.ds(o, 16)]
            dst16 = dstb[pl.ds(o, 16)]
            sv = plsc.load_gather(sb, [src16])
            wv = ewb[pl.ds(o, 16)]
            vb[0, pl.ds(g * 16, 16)] = sv * wv
            gb[0, pl.ds(g * 16, 16)] = plsc.load_gather(bb, [dst16])
        pltpu.sync_copy(vb.at[0], pp_sh.at[gb.at[0]], add=True)
        return carry

    lax.fori_loop(0, NCH, chunk, 0)
    plsc.subcore_barrier()

    @pl.when(s == 0)
    def _():
        pltpu.sync_copy(pp_sh, out_hbm.at[c])


def _k5_body(pp_ref, cnt_ref, b2_ref, o_ref):
    acc = pp_ref[...]
    t = acc[0:1, :] + acc[1:2, :]
    cnt = cnt_ref[...]
    o_ref[...] = t / jnp.maximum(cnt * cnt, 1.0) + b2_ref[...]


def _k5(pp, cnt, b2r):
    return pl.pallas_call(
        _k5_body,
        out_shape=jax.ShapeDtypeStruct((1, G), jnp.float32),
    )(pp, cnt, b2r)


def kernel(x, edge_index, edge_weight, batch, W1, b1, W2, b2):
    src = edge_index[0].astype(jnp.int32)
    dst = edge_index[1].astype(jnp.int32)
    batch1d = batch.astype(jnp.int32)
    batch2d = batch1d.reshape(N, 1)
    src_r = src.reshape(NW, NCH, CH)
    dst_r = dst.reshape(NW, NCH, CH)
    srcf = src.reshape(NW, EPW)
    dstf = dst.reshape(NW, EPW)
    ew_r = edge_weight.reshape(NW, EPW)
    b1r = b1.reshape(1, D)
    b2r = b2.reshape(1, 1)

    agg2 = _k2(x, src_r, dst_r, ew_r)
    s, cnt = _k3(agg2[0], agg2[1], batch2d, b1r, W1, W2)
    pp = _k4(s.reshape(N), srcf, dstf, ew_r, batch1d)
    p = _k5(pp, cnt, b2r)
    return p.reshape(G, 1)


# speedup vs baseline: 2.0980x; 1.0415x over previous
import functools

import jax
import jax.numpy as jnp
from jax import lax
from jax.experimental import pallas as pl
from jax.experimental.pallas import tpu as pltpu
from jax.experimental.pallas import tpu_sc as plsc

N = 10000
E = 320000
D = 128
G = 64
NW = 32
EPW = E // NW
CH = 80
NCH = EPW // CH
RPT = N // 16
BLK = 2000
NBLK = N // BLK

def _sc_mesh():
    return plsc.VectorSubcoreMesh(core_axis_name="c", subcore_axis_name="s",
                                  num_cores=2, num_subcores=16)


@functools.cache
def _k2_kernel():
    return pl.kernel(
        _k2_body,
        out_type=jax.ShapeDtypeStruct((2, N, D), jnp.float32),
        mesh=_sc_mesh(),
        compiler_params=pltpu.CompilerParams(use_tc_tiling_on_sc=False, needs_layout_passes=False),
        scratch_types=[
            pltpu.VMEM((2, CH), jnp.int32),
            pltpu.VMEM((2, CH), jnp.int32),
            pltpu.VMEM((2 * CH,), jnp.float32),
            pltpu.VMEM((CH, D), jnp.float32),
            pltpu.VMEM((CH, D), jnp.float32),
            pltpu.VMEM((CH, D), jnp.float32),
            pltpu.VMEM((CH, D), jnp.float32),
            pltpu.VMEM_SHARED((N, D), jnp.float32),
        ] + [pltpu.SemaphoreType.DMA] * 10,
    )


def _k2(*args):
    return _k2_kernel()(*args)


def _k2_body(y_hbm, src_hbm, dst_hbm, ew_hbm, out_hbm,
             srcr, dstr, ewr, rb0, rb1, ob0, ob1, agg_sh,
             gs0, gs1, ss0, ss1, ps0, ps1, pd0, pd1, pe0, pe1):
    c = lax.axis_index("c")
    s = lax.axis_index("s")
    w = s * 2 + c

    z16 = jnp.zeros((16,), jnp.float32)

    def zrow(i, carry):
        for f in range(D // 16):
            rb0[i, pl.ds(f * 16, 16)] = z16
        return carry

    lax.fori_loop(0, 25, zrow, 0)
    for j in range(RPT // 25):
        pltpu.sync_copy(rb0.at[pl.ds(0, 25)],
                        agg_sh.at[pl.ds(s * RPT + j * 25, 25)])
    plsc.subcore_barrier()

    rbufs = (rb0, rb1)
    obufs = (ob0, ob1)
    gsems = (gs0, gs1)
    ssems = (ss0, ss1)
    srcsems = (ps0, ps1)
    dstsems = (pd0, pd1)
    ewsems = (pe0, pe1)

    for b in range(2):
        pltpu.async_copy(src_hbm.at[w, b], srcr.at[b], srcsems[b])
        pltpu.async_copy(dst_hbm.at[w, b], dstr.at[b], dstsems[b])
        pltpu.async_copy(ew_hbm.at[w, pl.ds(b * CH, CH)],
                         ewr.at[pl.ds(b * CH, CH)], ewsems[b])
    pltpu.make_async_copy(src_hbm.at[w, 0], srcr.at[0], srcsems[0]).wait()
    pltpu.async_copy(y_hbm.at[srcr.at[0]], rb0, gs0)

    def body(i, b):
        rb, ob = rbufs[b], obufs[b]
        pltpu.make_async_copy(y_hbm.at[srcr.at[b]], rb, gsems[b]).wait()
        @pl.when(i + 2 < NCH)
        def _():
            pltpu.async_copy(src_hbm.at[w, i + 2], srcr.at[b], srcsems[b])
        @pl.when(i + 1 < NCH)
        def _():
            pltpu.make_async_copy(src_hbm.at[w, i + 1], srcr.at[1 - b],
                                  srcsems[1 - b]).wait()
            pltpu.async_copy(y_hbm.at[srcr.at[1 - b]], rbufs[1 - b],
                             gsems[1 - b])
        @pl.when(i >= 2)
        def _():
            pltpu.make_async_copy(ob, agg_sh.at[dstr.at[b]], ssems[b]).wait()
            pltpu.async_copy(dst_hbm.at[w, i], dstr.at[b], dstsems[b])
        pltpu.make_async_copy(ew_hbm.at[w, pl.ds(b * CH, CH)],
                              ewr.at[pl.ds(b * CH, CH)], ewsems[b]).wait()

        def scale(e):
            idx = lax.broadcast(b * CH + e, (16,))
            wv = plsc.load_gather(ewr, [idx])
            for f in range(D // 16):
                ob[e, pl.ds(f * 16, 16)] = rb[e, pl.ds(f * 16, 16)] * wv

        plsc.parallel_loop(0, CH, 1, unroll=4)(scale)
        pltpu.make_async_copy(dst_hbm.at[w, i], dstr.at[b], dstsems[b]).wait()
        pltpu.async_copy(ob, agg_sh.at[dstr.at[b]], ssems[b], add=True)
        @pl.when(i + 2 < NCH)
        def _():
            pltpu.async_copy(ew_hbm.at[w, pl.ds((i + 2) * CH, CH)],
                             ewr.at[pl.ds(b * CH, CH)], ewsems[b])

    def outer(j, carry):
        for b in range(2):
            body(2 * j + b, b)
        return carry

    lax.fori_loop(0, NCH // 2, outer, 0)
    body(jnp.int32(NCH - 1), 0)
    pltpu.make_async_copy(ob1, agg_sh.at[dstr.at[1]], ssems[1]).wait()
    pltpu.make_async_copy(ob0, agg_sh.at[dstr.at[0]], ssems[0]).wait()
    plsc.subcore_barrier()
    pltpu.sync_copy(agg_sh.at[pl.ds(s * RPT, RPT)],
                    out_hbm.at[c, pl.ds(s * RPT, RPT)])


def _k3_body(a0_ref, a1_ref, b_ref, b1_ref, w1_ref, w2_ref, s_ref, cnt_ref):
    p = pl.program_id(0)
    oh = (b_ref[...] == lax.broadcasted_iota(jnp.int32, (1, G), 1)
          ).astype(jnp.float32)

    @pl.when(p == 0)
    def _():
        @pl.when(pl.program_id(1) == 0)
        def _():
            cnt_ref[...] = jnp.zeros_like(cnt_ref)

        cnt_ref[...] += jnp.sum(oh, axis=0, keepdims=True)

    @pl.when(p == 1)
    def _():
        n_each = lax.dot_general(oh, cnt_ref[...], (((1,), (1,)), ((), ())),
                                 preferred_element_type=jnp.float32,
                                 precision=lax.Precision.HIGHEST)
        ninv = 1.0 / jnp.maximum(n_each, 1.0)
        a = (a0_ref[...] + a1_ref[...]) * ninv
        h = jax.nn.sigmoid(
            lax.dot_general(a, w1_ref[...], (((1,), (1,)), ((), ())),
                            preferred_element_type=jnp.float32,
                            precision=lax.Precision.HIGHEST) + b1_ref[...])
        s_ref[...] = lax.dot_general(h, w2_ref[...], (((1,), (1,)), ((), ())),
                                     preferred_element_type=jnp.float32,
                                     precision=lax.Precision.HIGHEST)


def _k3(a0, a1, batch2d, b1r, w1, w2):
    return pl.pallas_call(
        _k3_body,
        grid=(2, NBLK),
        in_specs=[
            pl.BlockSpec((BLK, D), lambda p, j: (p * j, 0)),
            pl.BlockSpec((BLK, D), lambda p, j: (p * j, 0)),
            pl.BlockSpec((BLK, 1), lambda p, j: (j, 0)),
            pl.BlockSpec((1, D), lambda p, j: (0, 0)),
            pl.BlockSpec((D, D), lambda p, j: (0, 0)),
            pl.BlockSpec((1, D), lambda p, j: (0, 0)),
        ],
        out_specs=[
            pl.BlockSpec((BLK, 1), lambda p, j: (p * j, 0)),
            pl.BlockSpec((1, G), lambda p, j: (0, 0)),
        ],
        out_shape=[
            jax.ShapeDtypeStruct((N, 1), jnp.float32),
            jax.ShapeDtypeStruct((1, G), jnp.float32),
        ],
    )(a0, a1, batch2d, b1r, w1, w2)


@functools.cache
def _k4_kernel():
    return pl.kernel(
        _k4_body,
        out_type=jax.ShapeDtypeStruct((2, G), jnp.float32),
        mesh=_sc_mesh(),
        compiler_params=pltpu.CompilerParams(use_tc_tiling_on_sc=False, needs_layout_passes=False),
        scratch_types=[
            pltpu.VMEM((N,), jnp.float32),
            pltpu.VMEM((N,), jnp.int32),
            pltpu.VMEM((EPW,), jnp.int32),
            pltpu.VMEM((EPW,), jnp.int32),
            pltpu.VMEM((EPW,), jnp.float32),
            pltpu.VMEM((2, CH), jnp.float32),
            pltpu.VMEM((2, CH), jnp.int32),
            pltpu.VMEM_SHARED((G,), jnp.float32),
            pltpu.SemaphoreType.DMA,
            pltpu.SemaphoreType.DMA,
        ],
    )


def _k4(*args):
    return _k4_kernel()(*args)


def _k4_body(s_hbm, srcf_hbm, dstf_hbm, ew_hbm, batch_hbm, out_hbm,
             sb, bb, srcb, dstb, ewb, vb, gb, pp_sh, qs0, qs1):
    c = lax.axis_index("c")
    s = lax.axis_index("s")
    w = s * 2 + c
    pltpu.sync_copy(s_hbm, sb)
    pltpu.sync_copy(batch_hbm, bb)
    pltpu.sync_copy(srcf_hbm.at[w], srcb)
    pltpu.sync_copy(dstf_hbm.at[w], dstb)
    pltpu.sync_copy(ew_hbm.at[w], ewb)

    z16 = jnp.zeros((16,), jnp.float32)
    for g in range(G // 16):
        vb[0, pl.ds(g * 16, 16)] = z16

    @pl.when(s == 0)
    def _():
        pltpu.sync_copy(vb.at[0, pl.ds(0, G)], pp_sh)

    plsc.subcore_barrier()

    qsems = (qs0, qs1)

    def bins(i, p):
        @pl.when(i >= 2)
        def _():
            pltpu.make_async_copy(vb.at[p], pp_sh.at[gb.at[p]],
                                  qsems[p]).wait()
        def grp(g, carry2):
            o = i * CH + g * 16
            src16 = srcb[pl.ds(o, 16)]
            dst16 = dstb[pl.ds(o, 16)]
            sv = plsc.load_gather(sb, [src16])
            wv = ewb[pl.ds(o, 16)]
            vb[p, pl.ds(g * 16, 16)] = sv * wv
            gb[p, pl.ds(g * 16, 16)] = plsc.load_gather(bb, [dst16])
            return carry2

        lax.fori_loop(0, CH // 16, grp, 0)
        pltpu.async_copy(vb.at[p], pp_sh.at[gb.at[p]], qsems[p], add=True)

    def chunk(j, carry):
        for p in range(2):
            bins(2 * j + p, p)
        return carry

    lax.fori_loop(0, NCH // 2, chunk, 0)
    bins(jnp.int32(NCH - 1), 0)
    pltpu.make_async_copy(vb.at[1], pp_sh.at[gb.at[1]], qsems[1]).wait()
    pltpu.make_async_copy(vb.at[0], pp_sh.at[gb.at[0]], qsems[0]).wait()
    plsc.subcore_barrier()

    @pl.when(s == 0)
    def _():
        pltpu.sync_copy(pp_sh, out_hbm.at[c])


def _k5_body(pp_ref, cnt_ref, b2_ref, o_ref):
    acc = pp_ref[...]
    t = acc[0:1, :] + acc[1:2, :]
    cnt = cnt_ref[...]
    o_ref[...] = t / jnp.maximum(cnt * cnt, 1.0) + b2_ref[...]


def _k5(pp, cnt, b2r):
    return pl.pallas_call(
        _k5_body,
        out_shape=jax.ShapeDtypeStruct((1, G), jnp.float32),
    )(pp, cnt, b2r)


def kernel(x, edge_index, edge_weight, batch, W1, b1, W2, b2):
    src = edge_index[0].astype(jnp.int32)
    dst = edge_index[1].astype(jnp.int32)
    batch1d = batch.astype(jnp.int32)
    batch2d = batch1d.reshape(N, 1)
    src_r = src.reshape(NW, NCH, CH)
    dst_r = dst.reshape(NW, NCH, CH)
    srcf = src.reshape(NW, EPW)
    dstf = dst.reshape(NW, EPW)
    ew_r = edge_weight.reshape(NW, EPW)
    b1r = b1.reshape(1, D)
    b2r = b2.reshape(1, 1)

    agg2 = _k2(x, src_r, dst_r, ew_r)
    s, cnt = _k3(agg2[0], agg2[1], batch2d, b1r, W1, W2)
    pp = _k4(s.reshape(N), srcf, dstf, ew_r, batch1d)
    p = _k5(pp, cnt, b2r)
    return p.reshape(G, 1)
